# baseline (speedup 1.0000x reference)
import jax
import jax.numpy as jnp
from jax.experimental import pallas as pl

_K = 16
_GRP = 16
_RB = 8
_FOLDS = [(1, 64), (1, 32), (1, 16), (1, 8), (1, 4), (1, 2), (1, 1),
          (0, 4), (0, 2), (0, 1)]


def _oddeven_sort_pairs(n):
    pairs = []

    def merge(lo, m, r):
        step = r * 2
        if step < m:
            merge(lo, m, step)
            merge(lo + r, m, step)
            for i in range(lo + r, lo + m - r, step):
                pairs.append((i, i + r))
        else:
            pairs.append((lo, lo + r))

    def sortnet(lo, m):
        if m > 1:
            h = m // 2
            sortnet(lo, h)
            sortnet(lo + h, h)
            merge(lo, m, 1)

    sortnet(0, n)
    return pairs


_SORT_PAIRS = _oddeven_sort_pairs(_GRP)


def _cmpx(lst, i, j):
    a, b = lst[i], lst[j]
    lst[i] = jnp.maximum(a, b)
    lst[j] = jnp.minimum(a, b)


def _merge_keep_top16(state, other):
    v = [jnp.maximum(state[k], other[_K - 1 - k]) for k in range(_K)]
    for d in (8, 4, 2, 1):
        for i in range(_K):
            if not i & d:
                _cmpx(v, i, i + d)
    return v


def _merge_equal(a, b):
    d = len(a)
    v = a + b[::-1]
    dist = d
    while dist >= 1:
        for i in range(2 * d):
            if not i & dist:
                _cmpx(v, i, i + dist)
        dist //= 2
    return v


def _fold_sorted(state):
    for axis, shift in _FOLDS:
        rolled = [pltpu_roll(s, shift, axis) for s in state]
        if len(state) < _K:
            state = _merge_equal(state, rolled)
        else:
            state = _merge_keep_top16(state, rolled)
    return state


def pltpu_roll(arr, shift, axis):
    return jnp.roll(arr, shift, axis=axis)


def _fallback_topk_sum(x_ref, r, nchunks):
    neg = jnp.float32(-jnp.inf)
    init = [jnp.full((8, 128), neg, dtype=jnp.float32) for _ in range(_K)]
    ngroups = nchunks // _GRP

    def insert_body(g, state):
        ch = [x_ref[r, pl.ds((g * _GRP + j) * 8, 8), :] for j in range(_GRP)]
        for (i, j) in _SORT_PAIRS:
            _cmpx(ch, i, j)
        return _merge_keep_top16(state, ch)

    state = jax.lax.fori_loop(0, ngroups, insert_body, init)
    state = _fold_sorted(state)
    total = state[0]
    for k in range(1, _K):
        total = total + state[k]
    return total[0, 0]


def _row_topk_sum(x_ref, r, nchunks):
    n_par = 8
    neg = jnp.full((8, 128), -jnp.inf, jnp.float32)
    m1s = [neg] * n_par
    m2s = [neg] * n_par
    for j in range(nchunks):
        c = x_ref[r, pl.ds(j * 8, 8), :]
        p = j % n_par
        t = jnp.minimum(m1s[p], c)
        m1s[p] = jnp.maximum(m1s[p], c)
        m2s[p] = jnp.maximum(m2s[p], t)
    lists = list(zip(m1s, m2s))
    while len(lists) > 1:
        nxt = []
        for i in range(0, len(lists) - 1, 2):
            (a0, a1), (b0, b1) = lists[i], lists[i + 1]
            hi = jnp.maximum(a0, b0)
            lo = jnp.maximum(jnp.minimum(a0, b0), jnp.maximum(a1, b1))
            nxt.append((hi, lo))
        if len(lists) % 2:
            nxt.append(lists[-1])
        lists = nxt
    m1, m2 = lists[0]

    cand = _fold_sorted([m1, m2])
    t0_arr = cand[_K - 1]
    t0 = t0_arr[0, 0]

    one = jnp.float32(1.0)
    zero = jnp.float32(0.0)
    n_par = 8
    s_accs = [jnp.zeros((8, 128), jnp.float32) for _ in range(n_par)]
    n_accs = [jnp.zeros((8, 128), jnp.float32) for _ in range(n_par)]
    for j in range(nchunks):
        c = x_ref[r, pl.ds(j * 8, 8), :]
        gt = c > t0_arr
        p = j % n_par
        s_accs[p] = s_accs[p] + jnp.where(gt, c, zero)
        n_accs[p] = n_accs[p] + jnp.where(gt, one, zero)
    s_tot = s_accs[0]
    n_tot = n_accs[0]
    for p in range(1, n_par):
        s_tot = s_tot + s_accs[p]
        n_tot = n_tot + n_accs[p]
    s_gt = jnp.sum(s_tot)
    n_gt = jnp.sum(n_tot)

    certified = s_gt + t0 * (jnp.float32(_K) - n_gt)
    return jax.lax.cond(
        n_gt <= jnp.float32(_K - 1),
        lambda: certified,
        lambda: _fallback_topk_sum(x_ref, r, nchunks),
    )


def _rows_topk_sum_kernel(x_ref, o_ref):
    nchunks = x_ref.shape[1] // 8
    for r in range(x_ref.shape[0]):
        y = _row_topk_sum(x_ref, r, nchunks)
        o_ref[r] = jnp.full((8, 128), y, dtype=jnp.float32)


def kernel(x):
    b, c, h, w = x.shape
    n = b * c
    hw = h * w
    assert hw % (1024 * _GRP) == 0, "row length must be a multiple of 8*128*16"
    rb = next(d for d in (_RB, 4, 2, 1) if n % d == 0)
    rows = hw // 128
    xr = x.reshape(n, rows, 128)

    out = pl.pallas_call(
        _rows_topk_sum_kernel,
        grid=(n // rb,),
        in_specs=[pl.BlockSpec((rb, rows, 128), lambda i: (i, 0, 0))],
        out_specs=pl.BlockSpec((rb, 8, 128), lambda i: (i, 0, 0)),
        out_shape=jax.ShapeDtypeStruct((n, 8, 128), jnp.float32),
    )(xr)
    return out[:, 0, 0].reshape(b, c)


# speedup vs baseline: 1.6062x; 1.6062x over previous
import jax
import jax.numpy as jnp
from jax.experimental import pallas as pl

_K = 16
_GRP = 16
_RB = 8
_FOLDS = [(1, 64), (1, 32), (1, 16), (1, 8), (1, 4), (1, 2), (1, 1),
          (0, 4), (0, 2), (0, 1)]


def _oddeven_sort_pairs(n):
    pairs = []

    def merge(lo, m, r):
        step = r * 2
        if step < m:
            merge(lo, m, step)
            merge(lo + r, m, step)
            for i in range(lo + r, lo + m - r, step):
                pairs.append((i, i + r))
        else:
            pairs.append((lo, lo + r))

    def sortnet(lo, m):
        if m > 1:
            h = m // 2
            sortnet(lo, h)
            sortnet(lo + h, h)
            merge(lo, m, 1)

    sortnet(0, n)
    return pairs


_SORT_PAIRS = _oddeven_sort_pairs(_GRP)


def _cmpx(lst, i, j):
    a, b = lst[i], lst[j]
    lst[i] = jnp.maximum(a, b)
    lst[j] = jnp.minimum(a, b)


def _merge_keep_top16(state, other):
    v = [jnp.maximum(state[k], other[_K - 1 - k]) for k in range(_K)]
    for d in (8, 4, 2, 1):
        for i in range(_K):
            if not i & d:
                _cmpx(v, i, i + d)
    return v


def _merge_equal(a, b):
    d = len(a)
    v = list(a) + list(b)[::-1]
    dist = d
    while dist >= 1:
        for i in range(2 * d):
            if not i & dist:
                _cmpx(v, i, i + dist)
        dist //= 2
    return v


def _fold_sorted(state):
    for axis, shift in _FOLDS:
        rolled = [jnp.roll(s, shift, axis=axis) for s in state]
        if len(state) < _K:
            state = _merge_equal(state, rolled)
        else:
            state = _merge_keep_top16(state, rolled)
    return state


def _row_certified(x_ref, r, nchunks):
    n_par = 8
    neg = jnp.full((8, 128), -jnp.inf, jnp.float32)
    m1s = [neg] * n_par
    m2s = [neg] * n_par
    m3s = [neg] * n_par
    for j in range(nchunks):
        c = x_ref[r, pl.ds(j * 8, 8), :]
        p = j % n_par
        t = jnp.minimum(m1s[p], c)
        m1s[p] = jnp.maximum(m1s[p], c)
        u = jnp.minimum(m2s[p], t)
        m2s[p] = jnp.maximum(m2s[p], t)
        m3s[p] = jnp.maximum(m3s[p], u)
    lists = [(m1s[p], m2s[p], m3s[p], neg) for p in range(n_par)]
    while len(lists) > 1:
        nxt = []
        for i in range(0, len(lists), 2):
            nxt.append(_merge_equal(lists[i], lists[i + 1])[:4])
        lists = nxt
    cand = _fold_sorted(list(lists[0]))
    t0_arr = cand[_K - 1]
    t0 = t0_arr[0, 0]

    one = jnp.float32(1.0)
    zero = jnp.float32(0.0)
    s_accs = [jnp.zeros((8, 128), jnp.float32) for _ in range(n_par)]
    n_accs = [jnp.zeros((8, 128), jnp.float32) for _ in range(n_par)]
    for j in range(nchunks):
        c = x_ref[r, pl.ds(j * 8, 8), :]
        gt = c > t0_arr
        p = j % n_par
        s_accs[p] = s_accs[p] + jnp.where(gt, c, zero)
        n_accs[p] = n_accs[p] + jnp.where(gt, one, zero)
    s_tot = s_accs[0]
    n_tot = n_accs[0]
    for p in range(1, n_par):
        s_tot = s_tot + s_accs[p]
        n_tot = n_tot + n_accs[p]
    s_gt = jnp.sum(s_tot)
    n_gt = jnp.sum(n_tot)
    certified = s_gt + t0 * (jnp.float32(_K) - n_gt)
    return certified, n_gt


def _main_kernel(x_ref, y_ref, n_ref):
    nchunks = x_ref.shape[1] // 8
    for r in range(x_ref.shape[0]):
        y, n_gt = _row_certified(x_ref, r, nchunks)
        y_ref[r] = jnp.full((8, 128), y, dtype=jnp.float32)
        n_ref[r] = jnp.full((8, 128), n_gt, dtype=jnp.float32)


def _rescue_kernel(x_ref, o_ref):
    nchunks = x_ref.shape[1] // 8
    neg = jnp.float32(-jnp.inf)
    init = [jnp.full((8, 128), neg, dtype=jnp.float32) for _ in range(_K)]
    ngroups = nchunks // _GRP

    def insert_body(g, state):
        ch = [x_ref[0, pl.ds((g * _GRP + j) * 8, 8), :] for j in range(_GRP)]
        for (i, j) in _SORT_PAIRS:
            _cmpx(ch, i, j)
        return _merge_keep_top16(state, ch)

    state = jax.lax.fori_loop(0, ngroups, insert_body, init)
    state = _fold_sorted(state)
    total = state[0]
    for k in range(1, _K):
        total = total + state[k]
    o_ref[0] = total


def kernel(x):
    b, c, h, w = x.shape
    n = b * c
    hw = h * w
    assert hw % (1024 * _GRP) == 0, "row length must be a multiple of 8*128*16"
    rb = next(d for d in (_RB, 4, 2, 1) if n % d == 0)
    rows = hw // 128
    xr = x.reshape(n, rows, 128)

    y_t, n_t = pl.pallas_call(
        _main_kernel,
        grid=(n // rb,),
        in_specs=[pl.BlockSpec((rb, rows, 128), lambda i: (i, 0, 0))],
        out_specs=[pl.BlockSpec((rb, 8, 128), lambda i: (i, 0, 0)),
                   pl.BlockSpec((rb, 8, 128), lambda i: (i, 0, 0))],
        out_shape=[jax.ShapeDtypeStruct((n, 8, 128), jnp.float32),
                   jax.ShapeDtypeStruct((n, 8, 128), jnp.float32)],
    )(xr)
    y = y_t[:, 0, 0]
    n_gt = n_t[:, 0, 0]

    def rescue():
        out = pl.pallas_call(
            _rescue_kernel,
            grid=(n,),
            in_specs=[pl.BlockSpec((1, rows, 128), lambda i: (i, 0, 0))],
            out_specs=pl.BlockSpec((1, 8, 128), lambda i: (i, 0, 0)),
            out_shape=jax.ShapeDtypeStruct((n, 8, 128), jnp.float32),
        )(xr)
        return out[:, 0, 0]

    y = jax.lax.cond(jnp.any(n_gt > jnp.float32(_K - 1)), rescue, lambda: y)
    return y.reshape(b, c)
